# baseline (speedup 1.0000x reference)
import functools

import jax
import jax.numpy as jnp
from jax import lax
from jax.experimental import pallas as pl
from jax.experimental.pallas import tpu as pltpu
from jax.experimental.pallas import tpu_sc as plsc

_CH = 128
_TPC = 8
_L = 16
_RU = 16
_NG = 4
_PAD = _CH + 1


@functools.cache
def _build(B0, K, V, D, NC, NS):
    NW = NC * NS
    T2 = D // 8
    T0 = B0 // _CH
    NCHUNK = T0 // _TPC
    TASKS = K * NCHUNK
    TPW = TASKS // NW
    CPW = TPW * _TPC
    mesh = plsc.VectorSubcoreMesh(core_axis_name="c", subcore_axis_name="s")

    @functools.partial(
        pl.kernel,
        mesh=mesh,
        compiler_params=pltpu.CompilerParams(
            use_tc_tiling_on_sc=False, needs_layout_passes=False
        ),
        out_type=jax.ShapeDtypeStruct((K, T2, NCHUNK, _TPC, 8, _CH), jnp.float32),
        scratch_types=[
            pltpu.VMEM((CPW, _CH), jnp.int32),
            pltpu.VMEM((_NG, _CH, D), jnp.float32),
            pltpu.VMEM((_TPC, D, _PAD), jnp.float32),
            pltpu.VMEM((_TPC, D, _PAD), jnp.float32),
        ]
        + [pltpu.SemaphoreType.DMA] * (_NG + 2),
    )
    def gather(idx3_hbm, table_hbm, out_hbm, idx_v, g_v, s0_v, s1_v, *sems):
        wid = lax.axis_index("s") * NC + lax.axis_index("c")
        gsem = sems[:_NG]
        strips = (s0_v, s1_v)
        wsems = sems[_NG:]

        pltpu.sync_copy(idx3_hbm.at[wid], idx_v)

        iota = lax.iota(jnp.int32, _L)
        jv0 = iota
        jv1 = iota + _L
        tid0 = wid * TPW

        def fire_gather(g, p):
            pltpu.async_copy(table_hbm.at[idx_v.at[g]], g_v.at[p], gsem[p])

        def wait_gather(p):
            pltpu.make_async_copy(
                table_hbm.at[pl.ds(0, _CH)], g_v.at[p], gsem[p]
            ).wait()

        def transpose_chunk(p, strip, tc):
            tcv = jnp.full((_L,), tc, jnp.int32)

            def rows(i, bv):
                vs = [
                    (g_v[p, i * _RU + d, pl.ds(0, _L)],
                     g_v[p, i * _RU + d, pl.ds(_L, _L)])
                    for d in range(_RU)
                ]
                for d, (v0, v1) in enumerate(vs):
                    plsc.store_scatter(strip, [tcv, jv0, bv + d], v0)
                    plsc.store_scatter(strip, [tcv, jv1, bv + d], v1)
                return bv + _RU

            lax.fori_loop(0, _CH // _RU, rows, jnp.zeros((_L,), jnp.int32))

        def task(tl, sq, drain):
            tid = tid0 + tl
            k = tid // NCHUNK
            c = tid % NCHUNK

            @pl.when(drain)
            def _():
                for t2 in range(T2):
                    pltpu.make_async_copy(
                        strips[sq].at[:, pl.ds(8 * t2, 8), pl.ds(0, _CH)],
                        out_hbm.at[0, t2, 0],
                        wsems[sq],
                    ).wait()

            for tc in range(_TPC):
                g = tl * _TPC + tc
                p = tc % _NG
                wait_gather(p)
                transpose_chunk(p, strips[sq], tc)

                @pl.when(g + _NG < CPW)
                def _():
                    fire_gather(g + _NG, p)

            for t2 in range(T2):
                pltpu.async_copy(
                    strips[sq].at[:, pl.ds(8 * t2, 8), pl.ds(0, _CH)],
                    out_hbm.at[k, t2, c],
                    wsems[sq],
                )

        for p in range(_NG):
            fire_gather(p, p)

        def super_body(i, carry):
            task(2 * i, 0, i > 0)
            task(2 * i + 1, 1, i > 0)
            return carry

        lax.fori_loop(0, TPW // 2, super_body, 0)
        task(TPW - 1, 0, TPW > 2)

        for sq in range(2):
            for t2 in range(T2):
                pltpu.make_async_copy(
                    strips[sq].at[:, pl.ds(8 * t2, 8), pl.ds(0, _CH)],
                    out_hbm.at[0, t2, 0],
                    wsems[sq],
                ).wait()

    return gather


def kernel(indices, R):
    B0, K = indices.shape
    V, D = R.shape
    info = plsc.get_sparse_core_info()
    NC, NS = info.num_cores, info.num_subcores
    NW = NC * NS
    idx3 = indices.astype(jnp.int32).T.reshape(NW, (B0 * K) // (NW * _CH), _CH)
    out6 = _build(B0, K, V, D, NC, NS)(idx3, R)
    out5 = out6.reshape(K, D // 8, B0 // _CH, 8, _CH)
    return out5.transpose(2, 4, 0, 1, 3).reshape(B0, K, D)


# speedup vs baseline: 1.0672x; 1.0672x over previous
import functools

import jax
import jax.numpy as jnp
from jax import lax
from jax.experimental import pallas as pl
from jax.experimental.pallas import tpu as pltpu
from jax.experimental.pallas import tpu_sc as plsc

_CH = 128
_TPC = 8
_L = 16
_RU = 8
_NG = 4
_PAD = _CH + 1


@functools.cache
def _build(B0, K, V, D, NC, NS):
    NW = NC * NS
    T2 = D // 8
    T0 = B0 // _CH
    NCHUNK = T0 // _TPC
    TASKS = K * NCHUNK
    TPW = TASKS // NW
    CPW = TPW * _TPC
    mesh = plsc.VectorSubcoreMesh(core_axis_name="c", subcore_axis_name="s")

    @functools.partial(
        pl.kernel,
        mesh=mesh,
        compiler_params=pltpu.CompilerParams(
            use_tc_tiling_on_sc=False, needs_layout_passes=False
        ),
        out_type=jax.ShapeDtypeStruct((K, T2, NCHUNK, _TPC, 8, _CH), jnp.float32),
        scratch_types=[
            pltpu.VMEM((CPW, _CH), jnp.int32),
            pltpu.VMEM((_NG, _CH, D), jnp.float32),
            pltpu.VMEM((_TPC, D, _PAD), jnp.float32),
            pltpu.VMEM((_TPC, D, _PAD), jnp.float32),
        ]
        + [pltpu.SemaphoreType.DMA] * (_NG + 2),
    )
    def gather(idx3_hbm, table_hbm, out_hbm, idx_v, g_v, s0_v, s1_v, *sems):
        wid = lax.axis_index("s") * NC + lax.axis_index("c")
        gsem = sems[:_NG]
        strips = (s0_v, s1_v)
        wsems = sems[_NG:]

        pltpu.sync_copy(idx3_hbm.at[wid], idx_v)

        iota = lax.iota(jnp.int32, _L)
        jv0 = iota
        jv1 = iota + _L
        tid0 = wid * TPW

        def fire_gather(g, p):
            pltpu.async_copy(table_hbm.at[idx_v.at[g]], g_v.at[p], gsem[p])

        def wait_gather(p):
            pltpu.make_async_copy(
                table_hbm.at[pl.ds(0, _CH)], g_v.at[p], gsem[p]
            ).wait()

        def transpose_chunk(p, strip, tc):
            tcv = jnp.full((_L,), tc, jnp.int32)

            def rows(i, bv):
                vs = [
                    (g_v[p, i * _RU + d, pl.ds(0, _L)],
                     g_v[p, i * _RU + d, pl.ds(_L, _L)])
                    for d in range(_RU)
                ]
                for d, (v0, v1) in enumerate(vs):
                    plsc.store_scatter(strip, [tcv, jv0, bv + d], v0)
                    plsc.store_scatter(strip, [tcv, jv1, bv + d], v1)
                return bv + _RU

            lax.fori_loop(0, _CH // _RU, rows, jnp.zeros((_L,), jnp.int32))

        def task(tl, sq, drain):
            tid = tid0 + tl
            k = tid // NCHUNK
            c = tid % NCHUNK

            @pl.when(drain)
            def _():
                for t2 in range(T2):
                    pltpu.make_async_copy(
                        strips[sq].at[:, pl.ds(8 * t2, 8), pl.ds(0, _CH)],
                        out_hbm.at[0, t2, 0],
                        wsems[sq],
                    ).wait()

            for tc in range(_TPC):
                g = tl * _TPC + tc
                p = tc % _NG
                wait_gather(p)
                transpose_chunk(p, strips[sq], tc)

                @pl.when(g + _NG < CPW)
                def _():
                    fire_gather(g + _NG, p)

            for t2 in range(T2):
                pltpu.async_copy(
                    strips[sq].at[:, pl.ds(8 * t2, 8), pl.ds(0, _CH)],
                    out_hbm.at[k, t2, c],
                    wsems[sq],
                )

        for p in range(_NG):
            fire_gather(p, p)

        def super_body(i, carry):
            task(2 * i, 0, i > 0)
            task(2 * i + 1, 1, i > 0)
            return carry

        lax.fori_loop(0, TPW // 2, super_body, 0)
        task(TPW - 1, 0, TPW > 2)

        for sq in range(2):
            for t2 in range(T2):
                pltpu.make_async_copy(
                    strips[sq].at[:, pl.ds(8 * t2, 8), pl.ds(0, _CH)],
                    out_hbm.at[0, t2, 0],
                    wsems[sq],
                ).wait()

    return gather


def kernel(indices, R):
    B0, K = indices.shape
    V, D = R.shape
    info = plsc.get_sparse_core_info()
    NC, NS = info.num_cores, info.num_subcores
    NW = NC * NS
    idx3 = indices.astype(jnp.int32).T.reshape(NW, (B0 * K) // (NW * _CH), _CH)
    out6 = _build(B0, K, V, D, NC, NS)(idx3, R)
    out5 = out6.reshape(K, D // 8, B0 // _CH, 8, _CH)
    return out5.transpose(2, 4, 0, 1, 3).reshape(B0, K, D)
